# baseline (speedup 1.0000x reference)
import jax
import jax.numpy as jnp
from jax.experimental import pallas as pl
from jax.experimental.pallas import tpu as pltpu


def _body(m_ref, x_ref, o_ref):
    o_ref[...] = x_ref[...] * m_ref[0]


def kernel(input, mask):
    B, C, K, H, W = input.shape
    BC = B * C
    HW = H * W
    x = input.reshape(BC * K, HW)
    m = mask.reshape(BC, 1, HW)

    ROWS = 8
    n = (BC * K) // ROWS

    out = pl.pallas_call(
        _body,
        grid=(n,),
        in_specs=[
            pl.BlockSpec((1, 1, HW), lambda j: (j * ROWS // K, 0, 0)),
            pl.BlockSpec((ROWS, HW), lambda j: (j, 0)),
        ],
        out_specs=pl.BlockSpec((ROWS, HW), lambda j: (j, 0)),
        out_shape=jax.ShapeDtypeStruct((BC * K, HW), x.dtype),
        compiler_params=pltpu.CompilerParams(
            dimension_semantics=("arbitrary",),
        ),
    )(m, x)
    return out.reshape(B, C, K, H, W)


# speedup vs baseline: 1.0494x; 1.0494x over previous
import jax
import jax.numpy as jnp
from jax.experimental import pallas as pl
from jax.experimental.pallas import tpu as pltpu


def _body(m_ref, x_ref, o_ref):
    o_ref[...] = x_ref[...] * m_ref[0]


def kernel(input, mask):
    B, C, K, H, W = input.shape
    BC = B * C
    HW = H * W
    x = input.reshape(BC * K, HW)
    m = mask.reshape(BC, 1, HW)

    ROWS = 32
    n = (BC * K) // ROWS

    out = pl.pallas_call(
        _body,
        grid=(n,),
        in_specs=[
            pl.BlockSpec((1, 1, HW), lambda j: (j * ROWS // K, 0, 0)),
            pl.BlockSpec((ROWS, HW), lambda j: (j, 0)),
        ],
        out_specs=pl.BlockSpec((ROWS, HW), lambda j: (j, 0)),
        out_shape=jax.ShapeDtypeStruct((BC * K, HW), x.dtype),
        compiler_params=pltpu.CompilerParams(
            dimension_semantics=("arbitrary",),
        ),
    )(m, x)
    return out.reshape(B, C, K, H, W)


# speedup vs baseline: 1.0875x; 1.0363x over previous
import functools

import jax
import jax.numpy as jnp
from jax.experimental import pallas as pl
from jax.experimental.pallas import tpu as pltpu


def _pipeline_body(m_hbm, x_hbm, o_hbm, xbuf, mbuf, obuf, xsem, msem, osem,
                   *, rows, nbuf, k):
    j = pl.program_id(0)
    nj = pl.num_programs(0)

    def in_copies(jj, slot):
        return (
            pltpu.make_async_copy(
                x_hbm.at[pl.ds(jj * rows, rows)], xbuf.at[slot], xsem.at[slot]),
            pltpu.make_async_copy(
                m_hbm.at[pl.ds(jj * rows // k, 1)], mbuf.at[slot], msem.at[slot]),
        )

    def start_in(jj):
        for c in in_copies(jj, jax.lax.rem(jj, nbuf)):
            c.start()

    @pl.when(j == 0)
    def _():
        for b in range(nbuf - 1):
            start_in(b)

    @pl.when(j + nbuf - 1 < nj)
    def _():
        start_in(j + nbuf - 1)

    slot = jax.lax.rem(j, nbuf)

    @pl.when(j >= nbuf)
    def _():
        pltpu.make_async_copy(
            obuf.at[slot], o_hbm.at[pl.ds((j - nbuf) * rows, rows)],
            osem.at[slot]).wait()

    for c in in_copies(j, slot):
        c.wait()

    obuf[slot] = xbuf[slot] * mbuf[slot]
    pltpu.make_async_copy(
        obuf.at[slot], o_hbm.at[pl.ds(j * rows, rows)], osem.at[slot]).start()

    @pl.when(j == nj - 1)
    def _():
        last = nj - 1
        for jj in range(max(0, last - nbuf + 1), last + 1):
            pltpu.make_async_copy(
                obuf.at[jj % nbuf], o_hbm.at[pl.ds(jj * rows, rows)],
                osem.at[jj % nbuf]).wait()


def kernel(input, mask):
    B, C, K, H, W = input.shape
    BC = B * C
    HW = H * W
    x = input.reshape(BC * K, HW)
    m = mask.reshape(BC, HW)

    ROWS = 16
    NBUF = 4
    nchunks = (BC * K) // ROWS

    out = pl.pallas_call(
        functools.partial(_pipeline_body, rows=ROWS, nbuf=NBUF, k=K),
        grid=(nchunks,),
        in_specs=[
            pl.BlockSpec(memory_space=pl.ANY),
            pl.BlockSpec(memory_space=pl.ANY),
        ],
        out_specs=pl.BlockSpec(memory_space=pl.ANY),
        out_shape=jax.ShapeDtypeStruct((BC * K, HW), x.dtype),
        scratch_shapes=[
            pltpu.VMEM((NBUF, ROWS, HW), x.dtype),
            pltpu.VMEM((NBUF, 1, HW), x.dtype),
            pltpu.VMEM((NBUF, ROWS, HW), x.dtype),
            pltpu.SemaphoreType.DMA((NBUF,)),
            pltpu.SemaphoreType.DMA((NBUF,)),
            pltpu.SemaphoreType.DMA((NBUF,)),
        ],
        compiler_params=pltpu.CompilerParams(
            dimension_semantics=("arbitrary",),
        ),
    )(m, x)
    return out.reshape(B, C, K, H, W)


# speedup vs baseline: 4.2496x; 3.9077x over previous
import jax
import jax.numpy as jnp
from jax.experimental import pallas as pl
from jax.experimental.pallas import tpu as pltpu


def _body(m_ref, x_ref, o_ref):
    o_ref[...] = x_ref[...] * m_ref[...]


def kernel(input, mask):
    B, C, K, H, W = input.shape
    BC = B * C
    x = input.reshape(BC * K, H, W)
    m = mask.reshape(BC, H, W)

    ROWS = 32
    n = (BC * K) // ROWS

    out = pl.pallas_call(
        _body,
        grid=(n,),
        in_specs=[
            pl.BlockSpec((1, H, W), lambda j: (j * ROWS // K, 0, 0)),
            pl.BlockSpec((ROWS, H, W), lambda j: (j, 0, 0)),
        ],
        out_specs=pl.BlockSpec((ROWS, H, W), lambda j: (j, 0, 0)),
        out_shape=jax.ShapeDtypeStruct((BC * K, H, W), x.dtype),
        compiler_params=pltpu.CompilerParams(
            dimension_semantics=("arbitrary",),
        ),
    )(m, x)
    return out.reshape(B, C, K, H, W)
